# baseline (speedup 1.0000x reference)
import jax
import jax.numpy as jnp
from jax import lax
from jax.experimental import pallas as pl
from jax.experimental.pallas import tpu as pltpu
from jax.experimental.pallas import tpu_sc as plsc

KH = KW = 3
H = W = 224
NPIX = H * W
C = 96
K = KH * KW
RPP = 2 * K
D2 = 2 * C

NW = 32
PPW = NPIX // NW
CPX = 4
RB = CPX * RPP
NCH = PPW // CPX
NRING = 8


def _prep_body(off_ref, w_ref, idx_ref, clr_ref):
    k = pl.program_id(0)
    ki = (k // KW).astype(jnp.float32)
    kj = (k % KW).astype(jnp.float32)
    hh = lax.broadcasted_iota(jnp.int32, (H, W), 0).astype(jnp.float32)
    ww = lax.broadcasted_iota(jnp.int32, (H, W), 1).astype(jnp.float32)
    py = hh - 1.0 + ki + off_ref[0, 0]
    px = ww - 1.0 + kj + off_ref[0, 1]
    y0 = jnp.floor(py)
    x0 = jnp.floor(px)
    ly = py - y0
    lx = px - x0
    x1 = x0 + 1.0
    wx0 = 1.0 - lx
    wx1 = lx
    vx0 = ((x0 >= 0) & (x0 <= W - 1)).astype(jnp.float32)
    vx1 = ((x1 >= 0) & (x1 <= W - 1)).astype(jnp.float32)
    bxf = jnp.clip(x0, 0, W - 2)
    bx = bxf.astype(jnp.int32)
    gxL = wx0 * vx0 * (x0 == bxf) + wx1 * vx1 * (x1 == bxf)
    gxR = wx0 * vx0 * (x0 == bxf + 1.0) + wx1 * vx1 * (x1 == bxf + 1.0)
    w = w_ref[0]
    for a, (ycf, wy) in enumerate(((y0, 1.0 - ly), (y0 + 1.0, ly))):
        vy = ((ycf >= 0) & (ycf <= H - 1)).astype(jnp.float32)
        yc = jnp.clip(ycf, 0, H - 1).astype(jnp.int32)
        wY = w * wy * vy
        idx_ref[0, a] = yc * W + bx
        clr_ref[0, a, 0] = wY * gxL
        clr_ref[0, a, 1] = wY * gxR


def _prep_call(off, w):
    return pl.pallas_call(
        _prep_body,
        grid=(K,),
        in_specs=[
            pl.BlockSpec((1, 2, H, W), lambda k: (k, 0, 0, 0)),
            pl.BlockSpec((1, H, W), lambda k: (k, 0, 0)),
        ],
        out_specs=[
            pl.BlockSpec((1, 2, H, W), lambda k: (k, 0, 0, 0)),
            pl.BlockSpec((1, 2, 2, H, W), lambda k: (k, 0, 0, 0, 0)),
        ],
        out_shape=[
            jax.ShapeDtypeStruct((K, 2, H, W), jnp.int32),
            jax.ShapeDtypeStruct((K, 2, 2, H, W), jnp.float32),
        ],
    )(off, w)


_GDN = lax.GatherDimensionNumbers(
    offset_dims=(), collapsed_slice_dims=(0,), start_index_map=(0,))


def _splat(vec, lane):
    idx = jnp.full((16, 1), lane, jnp.int32)
    return lax.gather(vec, idx, dimension_numbers=_GDN, slice_sizes=(1,),
                      mode=lax.GatherScatterMode.PROMISE_IN_BOUNDS)


def _compute_chunk(rows, cf_all, base_w, out_v, out_off):
    for p in range(CPX):
        accs = [jnp.zeros((16,), jnp.float32) for _ in range(6)]
        w0 = cf_all[pl.ds(base_w + p * RPP, 16)]
        w1 = cf_all[pl.ds(base_w + p * RPP + 2, 16)]
        for j in range(RPP):
            rr = p * RPP + j
            wsp = _splat(w0, j) if j < 16 else _splat(w1, j - 2)
            cpair = plsc.bitcast(wsp, jnp.bfloat16)
            for g in range(6):
                v = rows[rr, pl.ds(g * 32, 32)]
                prod = v * cpair
                aL, aR = plsc.unpack(prod, format=plsc.PackFormat.INTERLEAVED)
                accs[g] = accs[g] + aL + aR
        for g in range(6):
            out_v[pl.ds(out_off + p * C + g * 16, 16)] = accs[g]


def _sc_body(t2_hbm, idx_hbm, cfw_hbm, out_hbm, idx_all, cf_all,
             rows, outs, gsems, osems):
    cid = lax.axis_index("c")
    sid = lax.axis_index("s")
    wid = sid * 2 + cid
    base_px = wid * PPW
    mwords = PPW * RPP

    pltpu.sync_copy(idx_hbm.at[pl.ds(pl.multiple_of(base_px * RPP, 8), mwords)], idx_all)
    pltpu.sync_copy(cfw_hbm.at[pl.ds(pl.multiple_of(base_px * RPP, 8), mwords)], cf_all)

    def gslice(ch):
        return idx_all.at[pl.ds(pl.multiple_of(ch * RB, 8), RB)]

    def gather(ch, b):
        pltpu.async_copy(t2_hbm.at[gslice(ch)], rows[b], gsems[b])

    def gwait(ch, b):
        pltpu.make_async_copy(t2_hbm.at[gslice(ch)], rows[b], gsems[b]).wait()

    def oslice(u):
        return out_hbm.at[pl.ds(pl.multiple_of((base_px + u * 2 * CPX) * C, 8),
                                2 * CPX * C)]

    for b in range(NRING):
        gather(b, b)

    def group(g, carry):
        for b in range(NRING):
            ch = g * NRING + b
            ub = b // 2
            ostage = b % 2
            u = 4 * g + ub
            gwait(ch, b)
            if ostage == 0:
                @pl.when(g > 0)
                def _():
                    pltpu.make_async_copy(
                        outs[ub], oslice(u - 4), osems[ub]).wait()

            _compute_chunk(rows[b], cf_all, ch * RB, outs[ub], ostage * CPX * C)
            if ostage == 1:
                pltpu.async_copy(outs[ub], oslice(u), osems[ub])

            @pl.when(ch + NRING < NCH)
            def _():
                gather(ch + NRING, b)
        return carry

    lax.fori_loop(0, NCH // NRING, group, 0)
    nunits = NCH // 2
    for t in range(4):
        u = nunits - 4 + t
        pltpu.make_async_copy(outs[t], oslice(u), osems[t]).wait()


@jax.jit
def _sc_call(t2, idx_f, cfw):
    mesh = plsc.VectorSubcoreMesh(core_axis_name="c", subcore_axis_name="s")
    f = pl.kernel(
        _sc_body,
        out_type=jax.ShapeDtypeStruct((NPIX * C,), jnp.float32),
        mesh=mesh,
        scratch_types=[
            pltpu.VMEM((PPW * RPP,), jnp.int32),
            pltpu.VMEM((PPW * RPP,), jnp.int32),
            [pltpu.VMEM((RB, D2), jnp.bfloat16) for _ in range(NRING)],
            [pltpu.VMEM((2 * CPX * C,), jnp.float32) for _ in range(4)],
            [pltpu.SemaphoreType.DMA for _ in range(NRING)],
            [pltpu.SemaphoreType.DMA for _ in range(4)],
        ],
        compiler_params=pltpu.CompilerParams(
            needs_layout_passes=False, use_tc_tiling_on_sc=False),
    )
    return f(t2, idx_f, cfw)


def kernel(input, offset, weight):
    x = input.reshape(C, NPIX)
    xT = jnp.transpose(x)
    xR = jnp.roll(xT, -1, axis=0)
    t2 = jnp.stack([xT, xR], axis=2).reshape(NPIX, D2).astype(jnp.bfloat16)
    idx2, clr = _prep_call(
        offset.reshape(K, 2, H, W), weight.reshape(K, H, W))
    idx_f = idx2.reshape(RPP, NPIX).T.reshape(-1)
    cfb = clr.reshape(RPP, 2, NPIX).transpose(2, 0, 1).astype(jnp.bfloat16)
    cfw = lax.bitcast_convert_type(cfb, jnp.int32).reshape(-1)
    outT = _sc_call(t2, idx_f, cfw)
    return outT.reshape(NPIX, C).T.reshape(1, C, H, W)


# speedup vs baseline: 1.0904x; 1.0904x over previous
import jax
import jax.numpy as jnp
from jax import lax
from jax.experimental import pallas as pl
from jax.experimental.pallas import tpu as pltpu
from jax.experimental.pallas import tpu_sc as plsc

KH = KW = 3
H = W = 224
NPIX = H * W
C = 96
K = KH * KW
RPP = 2 * K
D2 = 2 * C

NW = 32
PPW = NPIX // NW
CPX = 4
RB = CPX * RPP
NCH = PPW // CPX
NRING = 4


def _prep_body(off_ref, w_ref, idx_ref, clr_ref):
    k = pl.program_id(0)
    ki = (k // KW).astype(jnp.float32)
    kj = (k % KW).astype(jnp.float32)
    hh = lax.broadcasted_iota(jnp.int32, (H, W), 0).astype(jnp.float32)
    ww = lax.broadcasted_iota(jnp.int32, (H, W), 1).astype(jnp.float32)
    py = hh - 1.0 + ki + off_ref[0, 0]
    px = ww - 1.0 + kj + off_ref[0, 1]
    y0 = jnp.floor(py)
    x0 = jnp.floor(px)
    ly = py - y0
    lx = px - x0
    x1 = x0 + 1.0
    wx0 = 1.0 - lx
    wx1 = lx
    vx0 = ((x0 >= 0) & (x0 <= W - 1)).astype(jnp.float32)
    vx1 = ((x1 >= 0) & (x1 <= W - 1)).astype(jnp.float32)
    bxf = jnp.clip(x0, 0, W - 2)
    bx = bxf.astype(jnp.int32)
    gxL = wx0 * vx0 * (x0 == bxf) + wx1 * vx1 * (x1 == bxf)
    gxR = wx0 * vx0 * (x0 == bxf + 1.0) + wx1 * vx1 * (x1 == bxf + 1.0)
    w = w_ref[0]
    for a, (ycf, wy) in enumerate(((y0, 1.0 - ly), (y0 + 1.0, ly))):
        vy = ((ycf >= 0) & (ycf <= H - 1)).astype(jnp.float32)
        yc = jnp.clip(ycf, 0, H - 1).astype(jnp.int32)
        wY = w * wy * vy
        idx_ref[0, a] = yc * W + bx
        clr_ref[0, a, 0] = wY * gxL
        clr_ref[0, a, 1] = wY * gxR


def _prep_call(off, w):
    return pl.pallas_call(
        _prep_body,
        grid=(K,),
        in_specs=[
            pl.BlockSpec((1, 2, H, W), lambda k: (k, 0, 0, 0)),
            pl.BlockSpec((1, H, W), lambda k: (k, 0, 0)),
        ],
        out_specs=[
            pl.BlockSpec((1, 2, H, W), lambda k: (k, 0, 0, 0)),
            pl.BlockSpec((1, 2, 2, H, W), lambda k: (k, 0, 0, 0, 0)),
        ],
        out_shape=[
            jax.ShapeDtypeStruct((K, 2, H, W), jnp.int32),
            jax.ShapeDtypeStruct((K, 2, 2, H, W), jnp.float32),
        ],
    )(off, w)


_GDN = lax.GatherDimensionNumbers(
    offset_dims=(), collapsed_slice_dims=(0,), start_index_map=(0,))


def _splat(vec, lane):
    idx = jnp.full((16, 1), lane, jnp.int32)
    return lax.gather(vec, idx, dimension_numbers=_GDN, slice_sizes=(1,),
                      mode=lax.GatherScatterMode.PROMISE_IN_BOUNDS)


def _compute_chunk(rows, cf_all, base_w, out_v, out_off):
    for p in range(CPX):
        accs = [jnp.zeros((16,), jnp.float32) for _ in range(6)]
        w0 = cf_all[pl.ds(base_w + p * RPP, 16)]
        w1 = cf_all[pl.ds(base_w + p * RPP + 2, 16)]
        for j in range(RPP):
            rr = p * RPP + j
            wsp = _splat(w0, j) if j < 16 else _splat(w1, j - 2)
            cpair = plsc.bitcast(wsp, jnp.bfloat16)
            for g in range(6):
                v = rows[rr, pl.ds(g * 32, 32)]
                prod = v * cpair
                aL, aR = plsc.unpack(prod, format=plsc.PackFormat.INTERLEAVED)
                accs[g] = accs[g] + aL + aR
        for g in range(6):
            out_v[pl.ds(out_off + p * C + g * 16, 16)] = accs[g]


def _sc_body(t2_hbm, idx_hbm, cfw_hbm, out_hbm, idx_all, cf_all,
             rows, outs, gsems, osems):
    cid = lax.axis_index("c")
    sid = lax.axis_index("s")
    wid = sid * 2 + cid
    base_px = wid * PPW
    mwords = PPW * RPP

    pltpu.sync_copy(idx_hbm.at[pl.ds(pl.multiple_of(base_px * RPP, 8), mwords)], idx_all)
    pltpu.sync_copy(cfw_hbm.at[pl.ds(pl.multiple_of(base_px * RPP, 8), mwords)], cf_all)

    def gslice(ch):
        return idx_all.at[pl.ds(pl.multiple_of(ch * RB, 8), RB)]

    def gather(ch, b):
        pltpu.async_copy(t2_hbm.at[gslice(ch)], rows[b], gsems[b])

    def gwait(ch, b):
        pltpu.make_async_copy(t2_hbm.at[gslice(ch)], rows[b], gsems[b]).wait()

    def oslice(u):
        return out_hbm.at[pl.ds(pl.multiple_of((base_px + u * 2 * CPX) * C, 8),
                                2 * CPX * C)]

    for b in range(NRING):
        gather(b, b)

    def group(g, carry):
        for b in range(NRING):
            ch = g * NRING + b
            ub = b // 2
            ostage = b % 2
            u = 2 * g + ub
            gwait(ch, b)
            if ostage == 0:
                @pl.when(g > 0)
                def _():
                    pltpu.make_async_copy(
                        outs[ub], oslice(u - 2), osems[ub]).wait()

            _compute_chunk(rows[b], cf_all, ch * RB, outs[ub], ostage * CPX * C)
            if ostage == 1:
                pltpu.async_copy(outs[ub], oslice(u), osems[ub])

            @pl.when(ch + NRING < NCH)
            def _():
                gather(ch + NRING, b)
        return carry

    lax.fori_loop(0, NCH // NRING, group, 0)
    nunits = NCH // 2
    for t in range(2):
        u = nunits - 2 + t
        pltpu.make_async_copy(outs[t], oslice(u), osems[t]).wait()


@jax.jit
def _sc_call(t2, idx_f, cfw):
    mesh = plsc.VectorSubcoreMesh(core_axis_name="c", subcore_axis_name="s")
    f = pl.kernel(
        _sc_body,
        out_type=jax.ShapeDtypeStruct((NPIX * C,), jnp.float32),
        mesh=mesh,
        scratch_types=[
            pltpu.VMEM((PPW * RPP,), jnp.int32),
            pltpu.VMEM((PPW * RPP,), jnp.int32),
            [pltpu.VMEM((RB, D2), jnp.bfloat16) for _ in range(NRING)],
            [pltpu.VMEM((2 * CPX * C,), jnp.float32) for _ in range(2)],
            [pltpu.SemaphoreType.DMA for _ in range(NRING)],
            [pltpu.SemaphoreType.DMA for _ in range(2)],
        ],
        compiler_params=pltpu.CompilerParams(
            needs_layout_passes=False, use_tc_tiling_on_sc=False),
    )
    return f(t2, idx_f, cfw)


def kernel(input, offset, weight):
    x = input.reshape(C, NPIX)
    xT = jnp.transpose(x)
    xR = jnp.roll(xT, -1, axis=0)
    t2 = jnp.stack([xT, xR], axis=2).reshape(NPIX, D2).astype(jnp.bfloat16)
    idx2, clr = _prep_call(
        offset.reshape(K, 2, H, W), weight.reshape(K, H, W))
    idx_f = idx2.reshape(RPP, NPIX).T.reshape(-1)
    cfb = clr.reshape(RPP, 2, NPIX).transpose(2, 0, 1).astype(jnp.bfloat16)
    cfw = lax.bitcast_convert_type(cfb, jnp.int32).reshape(-1)
    outT = _sc_call(t2, idx_f, cfw)
    return outT.reshape(NPIX, C).T.reshape(1, C, H, W)
